# baseline (speedup 1.0000x reference)
import functools

import jax
import jax.numpy as jnp
from jax.experimental import pallas as pl
from jax.experimental.pallas import tpu as pltpu

THRESH = 50.0
KWTA = 16

T = 16
K = 1024
RED = 65536

KO = 8
OBLK = K // KO
KC = 8
CBLK = RED // KC


def _matmul_kernel(a_ref, w_ref, pot_ref):
    c = pl.program_id(1)

    @pl.when(c == 0)
    def _init():
        pot_ref[...] = jnp.zeros_like(pot_ref)

    acc = jax.lax.dot_general(
        a_ref[...], w_ref[...],
        dimension_numbers=(((1,), (1,)), ((), ())),
        preferred_element_type=jnp.float32,
        precision=jax.lax.Precision.HIGHEST,
    )
    pot_ref[...] += acc

    @pl.when(c == KC - 1)
    def _threshold():
        v = pot_ref[...]
        pot_ref[...] = jnp.where(v > THRESH, v, 0.0)


def _epilogue_kernel(pot_ref, out_ref):
    pot = pot_ref[...]
    spikes = (pot > 0.0).astype(jnp.float32)
    vmax = jnp.max(pot) * T
    totals = jnp.sum(pot + spikes * vmax, axis=0, keepdims=True)

    iota = jax.lax.broadcasted_iota(jnp.int32, (1, K), 1)
    mask = jnp.zeros((1, K), jnp.float32)
    work = totals
    for _ in range(KWTA):
        m = jnp.max(work)
        idx = jnp.min(jnp.where(work == m, iota, K))
        won = (m > 0.0).astype(jnp.float32)
        sel = (iota == idx)
        mask = mask + jnp.where(sel, won, 0.0)
        work = jnp.where(sel, -jnp.inf, work)

    out_ref[...] = spikes * mask


@jax.jit
def kernel(rec_field, W):
    A = rec_field.reshape(T, RED)
    Wm = W.reshape(K, RED)

    pot = pl.pallas_call(
        _matmul_kernel,
        grid=(KO, KC),
        in_specs=[
            pl.BlockSpec((T, CBLK), lambda o, c: (0, c)),
            pl.BlockSpec((OBLK, CBLK), lambda o, c: (o, c)),
        ],
        out_specs=pl.BlockSpec((T, OBLK), lambda o, c: (0, o)),
        out_shape=jax.ShapeDtypeStruct((T, K), jnp.float32),
    )(A, Wm)

    spikes_masked = pl.pallas_call(
        _epilogue_kernel,
        out_shape=jax.ShapeDtypeStruct((T, K), jnp.float32),
    )(pot)

    return spikes_masked.reshape(T, K, 1, 1)


# speedup vs baseline: 1.3702x; 1.3702x over previous
import functools

import jax
import jax.numpy as jnp
from jax.experimental import pallas as pl
from jax.experimental.pallas import tpu as pltpu

THRESH = 50.0
KWTA = 16

T = 16
K = 1024
RED = 65536

KO = 8
OBLK = K // KO
KC = 8
CBLK = RED // KC


def _matmul_kernel(a_ref, w_ref, pot_ref):
    c = pl.program_id(1)

    @pl.when(c == 0)
    def _init():
        pot_ref[...] = jnp.zeros_like(pot_ref)

    acc = jax.lax.dot_general(
        a_ref[...], w_ref[...],
        dimension_numbers=(((1,), (1,)), ((), ())),
        preferred_element_type=jnp.float32,
    )
    pot_ref[...] += acc

    @pl.when(c == KC - 1)
    def _threshold():
        v = pot_ref[...]
        pot_ref[...] = jnp.where(v > THRESH, v, 0.0)


def _epilogue_kernel(pot_ref, out_ref):
    pot = pot_ref[...]
    spikes = (pot > 0.0).astype(jnp.float32)
    vmax = jnp.max(pot) * T
    totals = jnp.sum(pot + spikes * vmax, axis=0, keepdims=True)

    iota = jax.lax.broadcasted_iota(jnp.int32, (1, K), 1)
    mask = jnp.zeros((1, K), jnp.float32)
    work = totals
    for _ in range(KWTA):
        m = jnp.max(work)
        idx = jnp.min(jnp.where(work == m, iota, K))
        won = (m > 0.0).astype(jnp.float32)
        sel = (iota == idx)
        mask = mask + jnp.where(sel, won, 0.0)
        work = jnp.where(sel, -jnp.inf, work)

    out_ref[...] = spikes * mask


@jax.jit
def kernel(rec_field, W):
    A = rec_field.reshape(T, RED)
    Wm = W.reshape(K, RED)

    pot = pl.pallas_call(
        _matmul_kernel,
        grid=(KO, KC),
        in_specs=[
            pl.BlockSpec((T, CBLK), lambda o, c: (0, c)),
            pl.BlockSpec((OBLK, CBLK), lambda o, c: (o, c)),
        ],
        out_specs=pl.BlockSpec((T, OBLK), lambda o, c: (0, o)),
        out_shape=jax.ShapeDtypeStruct((T, K), jnp.float32),
    )(A, Wm)

    spikes_masked = pl.pallas_call(
        _epilogue_kernel,
        out_shape=jax.ShapeDtypeStruct((T, K), jnp.float32),
    )(pot)

    return spikes_masked.reshape(T, K, 1, 1)


# speedup vs baseline: 1.4420x; 1.0524x over previous
import functools

import jax
import jax.numpy as jnp
from jax.experimental import pallas as pl
from jax.experimental.pallas import tpu as pltpu

THRESH = 50.0
KWTA = 16

T = 16
K = 1024
RED = 65536

KC = 32
CBLK = RED // KC


def _matmul_kernel(a_ref, w_ref, pot_ref):
    c = pl.program_id(0)

    @pl.when(c == 0)
    def _init():
        pot_ref[...] = jnp.zeros_like(pot_ref)

    acc = jax.lax.dot_general(
        a_ref[...], w_ref[...],
        dimension_numbers=(((1,), (1,)), ((), ())),
        preferred_element_type=jnp.float32,
    )
    pot_ref[...] += acc

    @pl.when(c == KC - 1)
    def _threshold():
        v = pot_ref[...]
        pot_ref[...] = jnp.where(v > THRESH, v, 0.0)


def _epilogue_kernel(pot_ref, out_ref):
    pot = pot_ref[...]
    spikes = (pot > 0.0).astype(jnp.float32)
    vmax = jnp.max(pot) * T
    totals = jnp.sum(pot + spikes * vmax, axis=0, keepdims=True)

    iota = jax.lax.broadcasted_iota(jnp.int32, (1, K), 1)
    mask = jnp.zeros((1, K), jnp.float32)
    work = totals
    for _ in range(KWTA):
        m = jnp.max(work)
        idx = jnp.min(jnp.where(work == m, iota, K))
        won = (m > 0.0).astype(jnp.float32)
        sel = (iota == idx)
        mask = mask + jnp.where(sel, won, 0.0)
        work = jnp.where(sel, -jnp.inf, work)

    out_ref[...] = spikes * mask


@jax.jit
def kernel(rec_field, W):
    A = rec_field.reshape(T, RED)
    Wm = W.reshape(K, RED)

    pot = pl.pallas_call(
        _matmul_kernel,
        grid=(KC,),
        in_specs=[
            pl.BlockSpec((T, CBLK), lambda c: (0, c)),
            pl.BlockSpec((K, CBLK), lambda c: (0, c)),
        ],
        out_specs=pl.BlockSpec((T, K), lambda c: (0, 0)),
        out_shape=jax.ShapeDtypeStruct((T, K), jnp.float32),
    )(A, Wm)

    spikes_masked = pl.pallas_call(
        _epilogue_kernel,
        out_shape=jax.ShapeDtypeStruct((T, K), jnp.float32),
    )(pot)

    return spikes_masked.reshape(T, K, 1, 1)


# speedup vs baseline: 3.9455x; 2.7361x over previous
import functools

import jax
import jax.numpy as jnp
from jax.experimental import pallas as pl
from jax.experimental.pallas import tpu as pltpu

THRESH = 50.0
KWTA = 16

T = 16
K = 1024
RED = 65536

H = 256
L = 256
HBLK = 8
KC = H // HBLK


def _matmul_kernel(a_ref, w_ref, pot_ref):
    c = pl.program_id(0)

    @pl.when(c == 0)
    def _init():
        pot_ref[...] = jnp.zeros_like(pot_ref)

    acc = jnp.zeros((T, K), jnp.float32)
    for hh in range(HBLK):
        acc += jax.lax.dot_general(
            a_ref[:, hh, :], w_ref[:, hh, :],
            dimension_numbers=(((1,), (1,)), ((), ())),
            preferred_element_type=jnp.float32,
        )
    pot_ref[...] += acc

    @pl.when(c == KC - 1)
    def _threshold():
        v = pot_ref[...]
        pot_ref[...] = jnp.where(v > THRESH, v, 0.0)


def _epilogue_kernel(pot_ref, out_ref):
    pot = pot_ref[...]
    spikes = (pot > 0.0).astype(jnp.float32)
    vmax = jnp.max(pot) * T
    totals = jnp.sum(pot + spikes * vmax, axis=0, keepdims=True)

    iota = jax.lax.broadcasted_iota(jnp.int32, (1, K), 1)
    mask = jnp.zeros((1, K), jnp.float32)
    work = totals
    for _ in range(KWTA):
        m = jnp.max(work)
        idx = jnp.min(jnp.where(work == m, iota, K))
        won = (m > 0.0).astype(jnp.float32)
        sel = (iota == idx)
        mask = mask + jnp.where(sel, won, 0.0)
        work = jnp.where(sel, -jnp.inf, work)

    out_ref[...] = spikes * mask


@jax.jit
def kernel(rec_field, W):
    A = jnp.squeeze(rec_field, 1)
    Wm = jnp.squeeze(W, 1)

    pot = pl.pallas_call(
        _matmul_kernel,
        grid=(KC,),
        in_specs=[
            pl.BlockSpec((T, HBLK, L), lambda c: (0, c, 0)),
            pl.BlockSpec((K, HBLK, L), lambda c: (0, c, 0)),
        ],
        out_specs=pl.BlockSpec((T, K), lambda c: (0, 0)),
        out_shape=jax.ShapeDtypeStruct((T, K), jnp.float32),
    )(A, Wm)

    spikes_masked = pl.pallas_call(
        _epilogue_kernel,
        out_shape=jax.ShapeDtypeStruct((T, K), jnp.float32),
    )(pot)

    return spikes_masked.reshape(T, K, 1, 1)


# speedup vs baseline: 3.9989x; 1.0135x over previous
import jax
import jax.numpy as jnp
from jax.experimental import pallas as pl
from jax.experimental.pallas import tpu as pltpu

THRESH = 50.0
KWTA = 16

T = 16
K = 1024
H = 256
L = 256
HBLK = 8
KC = H // HBLK


def _column_kernel(a_ref, w_ref, out_ref, pot_ref):
    c = pl.program_id(0)

    @pl.when(c == 0)
    def _init():
        pot_ref[...] = jnp.zeros_like(pot_ref)

    acc = jnp.zeros((T, K), jnp.float32)
    for hh in range(HBLK):
        acc += jax.lax.dot_general(
            a_ref[:, hh, :], w_ref[:, hh, :],
            dimension_numbers=(((1,), (1,)), ((), ())),
            preferred_element_type=jnp.float32,
        )
    pot_ref[...] += acc

    @pl.when(c == KC - 1)
    def _epilogue():
        raw = pot_ref[...]
        pot = jnp.where(raw > THRESH, raw, 0.0)
        spikes = (pot > 0.0).astype(jnp.float32)
        vmax = jnp.max(pot) * T
        totals = jnp.sum(pot + spikes * vmax, axis=0, keepdims=True)

        iota = jax.lax.broadcasted_iota(jnp.int32, (1, K), 1)
        mask = jnp.zeros((1, K), jnp.float32)
        work = totals
        for _ in range(KWTA):
            m = jnp.max(work)
            idx = jnp.min(jnp.where(work == m, iota, K))
            won = (m > 0.0).astype(jnp.float32)
            sel = (iota == idx)
            mask = mask + jnp.where(sel, won, 0.0)
            work = jnp.where(sel, -jnp.inf, work)

        out_ref[...] = spikes * mask


@jax.jit
def kernel(rec_field, W):
    A = jnp.squeeze(rec_field, 1)
    Wm = jnp.squeeze(W, 1)

    spikes_masked = pl.pallas_call(
        _column_kernel,
        grid=(KC,),
        in_specs=[
            pl.BlockSpec((T, HBLK, L), lambda c: (0, c, 0)),
            pl.BlockSpec((K, HBLK, L), lambda c: (0, c, 0)),
        ],
        out_specs=pl.BlockSpec((T, K), lambda c: (0, 0)),
        out_shape=jax.ShapeDtypeStruct((T, K), jnp.float32),
        scratch_shapes=[pltpu.VMEM((T, K), jnp.float32)],
    )(A, Wm)

    return spikes_masked.reshape(T, K, 1, 1)
